# baseline (speedup 1.0000x reference)
import functools

import jax
import jax.numpy as jnp
from jax import lax
from jax.experimental import pallas as pl
from jax.experimental.pallas import tpu as pltpu


def _dot(a, b):
    return jnp.dot(a, b, preferred_element_type=jnp.float32)


def _dgen(a, b, dims):
    return lax.dot_general(a, b, (dims, ((), ())),
                           preferred_element_type=jnp.float32)


def _mega_body(x_ref, h_ref, m_ref, dv_ref, de_ref, ei_ref, er_ref,
               k_ref, w1_ref, wa_ref, we_ref, w2_ref, wp_ref,
               out_ref,
               em_ref, ef_ref, kt_ref, g_ref, x2_ref, cacc_ref,
               *, inv_sqrt_d, n_items, tn):
    p = pl.program_id(0)
    i = pl.program_id(1)
    nt = pl.num_programs(1)

    @pl.when((p == 0) & (i == 0))
    def _init():
        em_ref[...] = jnp.zeros_like(em_ref)
        ef_ref[...] = jnp.zeros_like(ef_ref)
        cacc_ref[...] = jnp.zeros_like(cacc_ref)
        kt_ref[...] = k_ref[...].T

    @pl.when(p == 0)
    def _ph0():
        xw = _dot(x_ref[...], w1_ref[...])
        em_ref[...] += _dgen(h_ref[...], xw, ((0,), (0,)))

    @pl.when((p == 1) & (i == 0))
    def _scale_em():
        em_ref[...] = em_ref[...] * de_ref[...]

    @pl.when(p == 1)
    def _ph1():
        x1 = jnp.maximum(_dot(h_ref[...], em_ref[...]) * dv_ref[...], 0.0)
        t = _dot(x1, wa_ref[...])
        logits = _dot(t, kt_ref[...]) * inv_sqrt_d
        mx = jnp.max(logits, axis=1, keepdims=True)
        ex = jnp.exp(logits - mx)
        alpha = ex / jnp.sum(ex, axis=1, keepdims=True)
        aw = alpha * m_ref[...]
        ef_ref[...] += _dgen(aw, x1, ((0,), (0,)))

    @pl.when((p == 2) & (i == 0))
    def _mix():
        ef = ef_ref[...] + _dot(k_ref[...], we_ref[...])
        g_ref[...] = _dot(ef, w2_ref[...])

    @pl.when(p == 2)
    def _ph2():
        x2 = jnp.maximum(_dot(m_ref[...], g_ref[...]), 0.0)
        x2_ref[pl.ds(i * tn, tn), :] = x2
        cacc_ref[...] += jnp.sum(x2, axis=0, keepdims=True)

    @pl.when((p == 2) & (i == nt - 1))
    def _pool():
        x2 = x2_ref[...]
        ctx = cacc_ref[...] * (1.0 / n_items)
        v = _dgen(ctx, wp_ref[...], ((1,), (1,)))
        s = _dgen(x2, v, ((1,), (1,)))

        def _softmax_n(z):
            m = jnp.max(z, axis=0, keepdims=True)
            e_ = jnp.exp(z - m)
            return e_ / jnp.sum(e_, axis=0, keepdims=True)

        w = _softmax_n(s * ei_ref[...]) + _softmax_n(s * er_ref[...])
        out_ref[...] = jnp.sum(w * x2, axis=0, keepdims=True)


def kernel(x, H, K, M, D_v_inv, D_e_inv, E_intra, E_inter,
           W1, Wa, We, W2, Wp):
    n, d = x.shape
    e = H.shape[1]
    tn = 1000
    nt = n // tn
    f32 = jnp.float32

    de = D_e_inv.reshape(e, 1)
    dv = D_v_inv.reshape(n, 1)
    ei = E_intra.reshape(n, 1)
    er = E_inter.reshape(n, 1)

    last = nt - 1

    def _i_or_last(p, i):
        return (jnp.where(p == 0, i, last), 0)

    def _h_idx(p, i):
        return (jnp.where(p <= 1, i, last), 0)

    def _m_idx(p, i):
        return (jnp.where(p >= 1, i, last), 0)

    def _dv_idx(p, i):
        return (jnp.where(p == 1, i, last), 0)

    const2 = lambda p, i: (0, 0)

    out = pl.pallas_call(
        functools.partial(_mega_body, inv_sqrt_d=float(1.0 / (d ** 0.5)),
                          n_items=float(n), tn=tn),
        grid=(3, nt),
        in_specs=[
            pl.BlockSpec((tn, d), _i_or_last),
            pl.BlockSpec((tn, e), _h_idx),
            pl.BlockSpec((tn, e), _m_idx),
            pl.BlockSpec((tn, 1), _dv_idx),
            pl.BlockSpec((e, 1), const2),
            pl.BlockSpec((n, 1), const2),
            pl.BlockSpec((n, 1), const2),
            pl.BlockSpec((e, d), const2),
            pl.BlockSpec((d, d), const2),
            pl.BlockSpec((d, d), const2),
            pl.BlockSpec((d, d), const2),
            pl.BlockSpec((d, d), const2),
            pl.BlockSpec((d, d), const2),
        ],
        out_specs=pl.BlockSpec((1, d), const2),
        out_shape=jax.ShapeDtypeStruct((1, d), f32),
        scratch_shapes=[
            pltpu.VMEM((e, d), f32),
            pltpu.VMEM((e, d), f32),
            pltpu.VMEM((d, e), f32),
            pltpu.VMEM((e, d), f32),
            pltpu.VMEM((n, d), f32),
            pltpu.VMEM((1, d), f32),
        ],
        compiler_params=pltpu.CompilerParams(
            vmem_limit_bytes=100 * 1024 * 1024,
            dimension_semantics=('arbitrary', 'arbitrary'),
            fuse_transposed_lhs_in_matmul=True,
        ),
    )(x, H, M, dv, de, ei, er, K, W1, Wa, We, W2, Wp)

    return out.reshape(d)


# speedup vs baseline: 1.0169x; 1.0169x over previous
import functools

import jax
import jax.numpy as jnp
from jax import lax
from jax.experimental import pallas as pl
from jax.experimental.pallas import tpu as pltpu


def _dot(a, b):
    return jnp.dot(a, b, preferred_element_type=jnp.float32)


def _dgen(a, b, dims):
    return lax.dot_general(a, b, (dims, ((), ())),
                           preferred_element_type=jnp.float32)


def _mega_body(x_ref, h_ref, m_ref, dv_ref, de_ref, ei_ref, er_ref,
               k_ref, w1_ref, wa_ref, we_ref, w2_ref, wp_ref,
               out_ref,
               em_ref, ef_ref, kt_ref, g_ref, x2_ref, cacc_ref,
               *, inv_sqrt_d, n_items, tn):
    p = pl.program_id(0)
    i = pl.program_id(1)
    nt = pl.num_programs(1)

    @pl.when((p == 0) & (i == 0))
    def _init():
        em_ref[...] = jnp.zeros_like(em_ref)
        ef_ref[...] = jnp.zeros_like(ef_ref)
        cacc_ref[...] = jnp.zeros_like(cacc_ref)
        kt_ref[...] = k_ref[...].T

    @pl.when(p == 0)
    def _ph0():
        xw = _dot(x_ref[...], w1_ref[...])
        em_ref[...] += _dgen(h_ref[...], xw, ((0,), (0,)))

    @pl.when((p == 1) & (i == 0))
    def _scale_em():
        em_ref[...] = em_ref[...] * de_ref[...]

    @pl.when(p == 1)
    def _ph1():
        x1 = jnp.maximum(_dot(h_ref[...], em_ref[...]) * dv_ref[...], 0.0)
        t = _dot(x1, wa_ref[...])
        logits = _dot(t, kt_ref[...]) * inv_sqrt_d
        mx = jnp.max(logits, axis=1, keepdims=True)
        ex = jnp.exp(logits - mx)
        alpha = ex / jnp.sum(ex, axis=1, keepdims=True)
        aw = alpha * m_ref[...]
        ef_ref[...] += _dgen(aw, x1, ((0,), (0,)))

    @pl.when((p == 2) & (i == 0))
    def _mix():
        ef = ef_ref[...] + _dot(k_ref[...], we_ref[...])
        g_ref[...] = _dot(ef, w2_ref[...])

    @pl.when(p == 2)
    def _ph2():
        x2 = jnp.maximum(_dot(m_ref[...], g_ref[...]), 0.0)
        x2_ref[pl.ds(i * tn, tn), :] = x2
        cacc_ref[...] += jnp.sum(x2, axis=0, keepdims=True)

    @pl.when((p == 2) & (i == nt - 1))
    def _pool():
        x2 = x2_ref[...]
        ctx = cacc_ref[...] * (1.0 / n_items)
        v = _dgen(ctx, wp_ref[...], ((1,), (1,)))
        s = _dgen(x2, v, ((1,), (1,)))

        def _softmax_n(z):
            m = jnp.max(z, axis=0, keepdims=True)
            e_ = jnp.exp(z - m)
            return e_ / jnp.sum(e_, axis=0, keepdims=True)

        w = _softmax_n(s * ei_ref[...]) + _softmax_n(s * er_ref[...])
        out_ref[...] = jnp.sum(w * x2, axis=0, keepdims=True)


def kernel(x, H, K, M, D_v_inv, D_e_inv, E_intra, E_inter,
           W1, Wa, We, W2, Wp):
    n, d = x.shape
    e = H.shape[1]
    tn = 1000
    nt = n // tn
    f32 = jnp.float32

    de = D_e_inv.reshape(e, 1)
    dv = D_v_inv.reshape(n, 1)
    ei = E_intra.reshape(n, 1)
    er = E_inter.reshape(n, 1)

    last = nt - 1

    def _i_or_last(p, i):
        return (jnp.where(p == 0, i, last), 0)

    def _h_idx(p, i):
        return (jnp.where(p <= 1, i, last), 0)

    def _m_idx(p, i):
        return (jnp.where(p >= 1, i, last), 0)

    def _dv_idx(p, i):
        return (jnp.where(p == 1, i, last), 0)

    const2 = lambda p, i: (0, 0)

    out = pl.pallas_call(
        functools.partial(_mega_body, inv_sqrt_d=float(1.0 / (d ** 0.5)),
                          n_items=float(n), tn=tn),
        grid=(3, nt),
        in_specs=[
            pl.BlockSpec((tn, d), _i_or_last),
            pl.BlockSpec((tn, e), _h_idx),
            pl.BlockSpec((tn, e), _m_idx),
            pl.BlockSpec((tn, 1), _dv_idx),
            pl.BlockSpec((e, 1), const2),
            pl.BlockSpec((n, 1), const2),
            pl.BlockSpec((n, 1), const2),
            pl.BlockSpec((e, d), const2),
            pl.BlockSpec((d, d), const2),
            pl.BlockSpec((d, d), const2),
            pl.BlockSpec((d, d), const2),
            pl.BlockSpec((d, d), const2),
            pl.BlockSpec((d, d), const2),
        ],
        out_specs=pl.BlockSpec((1, d), const2),
        out_shape=jax.ShapeDtypeStruct((1, d), f32),
        scratch_shapes=[
            pltpu.VMEM((e, d), f32),
            pltpu.VMEM((e, d), f32),
            pltpu.VMEM((d, e), f32),
            pltpu.VMEM((e, d), f32),
            pltpu.VMEM((n, d), f32),
            pltpu.VMEM((1, d), f32),
        ],
        compiler_params=pltpu.CompilerParams(
            vmem_limit_bytes=100 * 1024 * 1024,
        ),
    )(x, H, M, dv, de, ei, er, K, W1, Wa, We, W2, Wp)

    return out.reshape(d)


# speedup vs baseline: 1.0422x; 1.0249x over previous
import functools

import jax
import jax.numpy as jnp
from jax import lax
from jax.experimental import pallas as pl
from jax.experimental.pallas import tpu as pltpu


def _dot(a, b):
    return jnp.dot(a, b, preferred_element_type=jnp.float32)


def _dgen(a, b, dims):
    return lax.dot_general(a, b, (dims, ((), ())),
                           preferred_element_type=jnp.float32)


def _mega_body(x_ref, h_ref, m_ref, dv_ref, de_ref, ei_ref, er_ref,
               k_ref, w1_ref, wa_ref, we_ref, w2_ref, wp_ref,
               out_ref,
               em_ref, ef_ref, kt_ref, g_ref, x2_ref, cacc_ref,
               *, inv_sqrt_d, n_items, tn):
    p = pl.program_id(0)
    i = pl.program_id(1)
    nt = pl.num_programs(1)

    @pl.when((p == 0) & (i == 0))
    def _init():
        em_ref[...] = jnp.zeros_like(em_ref)
        ef_ref[...] = jnp.zeros_like(ef_ref)
        cacc_ref[...] = jnp.zeros_like(cacc_ref)
        kt_ref[...] = k_ref[...].T

    @pl.when(p == 0)
    def _ph0():
        xw = _dot(x_ref[...], w1_ref[...])
        em_ref[...] += _dgen(h_ref[...], xw, ((0,), (0,)))

    @pl.when((p == 1) & (i == 0))
    def _scale_em():
        em_ref[...] = em_ref[...] * de_ref[...]

    @pl.when(p == 1)
    def _ph1():
        x1 = jnp.maximum(_dot(h_ref[...], em_ref[...]) * dv_ref[...], 0.0)
        t = _dot(x1, wa_ref[...])
        logits = _dot(t, kt_ref[...]) * inv_sqrt_d
        mx = jnp.max(logits, axis=1, keepdims=True)
        ex = jnp.exp(logits - mx)
        alpha = ex / jnp.sum(ex, axis=1, keepdims=True)
        aw = alpha * m_ref[...]
        ef_ref[...] += _dgen(aw, x1, ((0,), (0,)))

    @pl.when((p == 2) & (i == 0))
    def _mix():
        ef = ef_ref[...] + _dot(k_ref[...], we_ref[...])
        g_ref[...] = _dot(ef, w2_ref[...])

    @pl.when(p == 2)
    def _ph2():
        x2 = jnp.maximum(_dot(m_ref[...], g_ref[...]), 0.0)
        x2_ref[pl.ds(i * tn, tn), :] = x2
        cacc_ref[...] += jnp.sum(x2, axis=0, keepdims=True)

    @pl.when((p == 2) & (i == nt - 1))
    def _pool():
        x2 = x2_ref[...]
        ctx = cacc_ref[...] * (1.0 / n_items)
        v = _dgen(ctx, wp_ref[...], ((1,), (1,)))
        s = _dgen(x2, v, ((1,), (1,)))

        def _softmax_n(z):
            m = jnp.max(z, axis=0, keepdims=True)
            e_ = jnp.exp(z - m)
            return e_ / jnp.sum(e_, axis=0, keepdims=True)

        w = _softmax_n(s * ei_ref[...]) + _softmax_n(s * er_ref[...])
        out_ref[...] = jnp.sum(w * x2, axis=0, keepdims=True)


def kernel(x, H, K, M, D_v_inv, D_e_inv, E_intra, E_inter,
           W1, Wa, We, W2, Wp):
    n, d = x.shape
    e = H.shape[1]
    tn = 2000
    nt = n // tn
    f32 = jnp.float32

    de = D_e_inv.reshape(e, 1)
    dv = D_v_inv.reshape(n, 1)
    ei = E_intra.reshape(n, 1)
    er = E_inter.reshape(n, 1)

    last = nt - 1

    def _i_or_last(p, i):
        return (jnp.where(p == 0, i, last), 0)

    def _h_idx(p, i):
        return (jnp.where(p <= 1, i, last), 0)

    def _m_idx(p, i):
        return (jnp.where(p >= 1, i, last), 0)

    def _dv_idx(p, i):
        return (jnp.where(p == 1, i, last), 0)

    const2 = lambda p, i: (0, 0)

    out = pl.pallas_call(
        functools.partial(_mega_body, inv_sqrt_d=float(1.0 / (d ** 0.5)),
                          n_items=float(n), tn=tn),
        grid=(3, nt),
        in_specs=[
            pl.BlockSpec((tn, d), _i_or_last),
            pl.BlockSpec((tn, e), _h_idx),
            pl.BlockSpec((tn, e), _m_idx),
            pl.BlockSpec((tn, 1), _dv_idx),
            pl.BlockSpec((e, 1), const2),
            pl.BlockSpec((n, 1), const2),
            pl.BlockSpec((n, 1), const2),
            pl.BlockSpec((e, d), const2),
            pl.BlockSpec((d, d), const2),
            pl.BlockSpec((d, d), const2),
            pl.BlockSpec((d, d), const2),
            pl.BlockSpec((d, d), const2),
            pl.BlockSpec((d, d), const2),
        ],
        out_specs=pl.BlockSpec((1, d), const2),
        out_shape=jax.ShapeDtypeStruct((1, d), f32),
        scratch_shapes=[
            pltpu.VMEM((e, d), f32),
            pltpu.VMEM((e, d), f32),
            pltpu.VMEM((d, e), f32),
            pltpu.VMEM((e, d), f32),
            pltpu.VMEM((n, d), f32),
            pltpu.VMEM((1, d), f32),
        ],
        compiler_params=pltpu.CompilerParams(
            vmem_limit_bytes=100 * 1024 * 1024,
        ),
    )(x, H, M, dv, de, ei, er, K, W1, Wa, We, W2, Wp)

    return out.reshape(d)
